# Initial kernel scaffold; baseline (speedup 1.0000x reference)
import jax
import jax.numpy as jnp
from jax.experimental import pallas as pl


def kernel(x, tok_embed, ln_weight, ln_bias):
    raise NotImplementedError("write your pallas kernel here")


# speedup vs baseline: 1.8794x; 1.8794x over previous
import functools

import jax
import jax.numpy as jnp
from jax import lax
from jax.experimental import pallas as pl
from jax.experimental.pallas import tpu as pltpu
from jax.experimental.pallas import tpu_sc as plsc

EPS = 1e-5
D = 256
CHUNK = 128


def _ln_table_body(t_ref, w_ref, b_ref, o_ref):
    t = t_ref[...]
    mean = jnp.mean(t, axis=1, keepdims=True)
    c = t - mean
    var = jnp.mean(c * c, axis=1, keepdims=True)
    o_ref[...] = c * lax.rsqrt(var + EPS) * w_ref[...] + b_ref[...]


def _normalize_table(tok_embed, ln_weight, ln_bias):
    v = tok_embed.shape[0]
    vpad = (v + 7) // 8 * 8
    t = jnp.zeros((vpad, D), tok_embed.dtype).at[:v].set(tok_embed)
    return pl.pallas_call(
        _ln_table_body,
        out_shape=jax.ShapeDtypeStruct((vpad, D), jnp.float32),
    )(t, ln_weight.reshape(1, D), ln_bias.reshape(1, D))


def _make_gather(num_chunks, nc, ns):
    nw = nc * ns
    b_per_w = num_chunks * CHUNK
    mesh = plsc.VectorSubcoreMesh(core_axis_name="c", subcore_axis_name="s")

    @functools.partial(
        pl.kernel,
        mesh=mesh,
        out_type=jax.ShapeDtypeStruct((nw * b_per_w, D), jnp.float32),
        scratch_types=[
            pltpu.VMEM((num_chunks, CHUNK), jnp.int32),
            pltpu.VMEM((CHUNK, D), jnp.float32),
            pltpu.VMEM((CHUNK, D), jnp.float32),
            pltpu.SemaphoreType.DMA,
            pltpu.SemaphoreType.DMA,
            pltpu.SemaphoreType.DMA,
            pltpu.SemaphoreType.DMA,
        ],
    )
    def gather(tab_hbm, idx_hbm, out_hbm, idx_v, buf0, buf1, g0, g1, o0, o1):
        wid = lax.axis_index("s") * nc + lax.axis_index("c")
        base = wid * b_per_w
        pltpu.sync_copy(idx_hbm.at[wid], idx_v)
        bufs = (buf0, buf1)
        gsems = (g0, g1)
        osems = (o0, o1)

        def out_slice(c):
            return out_hbm.at[pl.ds(base + c * CHUNK, CHUNK)]

        pltpu.async_copy(tab_hbm.at[idx_v.at[0]], buf0, g0)

        def loop_body(c0, _):
            for s in range(2):
                cc = 2 * c0 + s

                @pl.when(cc + 1 < num_chunks)
                def _():
                    @pl.when(cc >= 1)
                    def _():
                        pltpu.make_async_copy(
                            bufs[1 - s], out_slice(cc - 1), osems[1 - s]
                        ).wait()

                    pltpu.async_copy(
                        tab_hbm.at[idx_v.at[cc + 1]], bufs[1 - s], gsems[1 - s]
                    )

                pltpu.make_async_copy(
                    tab_hbm.at[idx_v.at[cc]], bufs[s], gsems[s]
                ).wait()
                pltpu.async_copy(bufs[s], out_slice(cc), osems[s])
            return 0

        lax.fori_loop(0, num_chunks // 2, loop_body, 0, unroll=False)
        for cc in (num_chunks - 2, num_chunks - 1):
            s = cc % 2
            pltpu.make_async_copy(bufs[s], out_slice(cc), osems[s]).wait()

    return gather


def kernel(x, tok_embed, ln_weight, ln_bias):
    info = plsc.get_sparse_core_info()
    nc, ns = info.num_cores, info.num_subcores
    nw = nc * ns
    b = x.size
    num_chunks = b // (nw * CHUNK)
    assert num_chunks * nw * CHUNK == b and num_chunks % 2 == 0

    tab = _normalize_table(tok_embed, ln_weight, ln_bias)
    idx = x.reshape(nw, num_chunks, CHUNK)
    out = _make_gather(num_chunks, nc, ns)(tab, idx)
    return out.reshape(*x.shape, D)
